# baseline (speedup 1.0000x reference)
import functools

import jax
import jax.numpy as jnp
from jax import lax
from jax.experimental import pallas as pl
from jax.experimental.pallas import tpu as pltpu
from jax.experimental.pallas import tpu_sc as plsc

B = 128
L = 256
F_LAB = 512
D = 1024
P = 8
NEG = 32
TEXT_DIM = L * 128

NKB = 32
KBLK = TEXT_DIM // NKB


def _label_mlp_body(lab_ref, w1_ref, b1_ref, w2_ref, b2_ref, out_ref):
    h = jnp.dot(lab_ref[...].astype(jnp.bfloat16),
                w1_ref[...].astype(jnp.bfloat16),
                preferred_element_type=jnp.float32)
    h = jnp.maximum(h + b1_ref[...], 0.0)
    o = jnp.dot(h.astype(jnp.bfloat16), w2_ref[...].astype(jnp.bfloat16),
                preferred_element_type=jnp.float32)
    o = jnp.maximum(o + b2_ref[...], 0.0)
    n = jnp.sqrt(jnp.sum(o * o, axis=-1, keepdims=True))
    out_ref[...] = o / jnp.maximum(n, 1e-12)


def _text_mlp_body(x_ref, w1a_ref, w1b_ref, w1c_ref, w1d_ref,
                   b1_ref, w2_ref, b2_ref, out_ref, acc_ref):
    k = pl.program_id(0)

    @pl.when(k == 0)
    def _init():
        acc_ref[...] = jnp.zeros_like(acc_ref)

    xb = x_ref[...].astype(jnp.bfloat16)
    DQ = D // 4
    for j, wr in enumerate((w1a_ref, w1b_ref, w1c_ref, w1d_ref)):
        acc_ref[:, j * DQ:(j + 1) * DQ] += jnp.dot(
            xb, wr[...].astype(jnp.bfloat16),
            preferred_element_type=jnp.float32)

    @pl.when(k == NKB - 1)
    def _fin():
        h = jnp.maximum(acc_ref[...] + b1_ref[...], 0.0)
        o = jnp.dot(h, w2_ref[...], preferred_element_type=jnp.float32)
        o = jnp.maximum(o + b2_ref[...], 0.0)
        n = jnp.sqrt(jnp.sum(o * o, axis=-1, keepdims=True))
        out_ref[...] = o / jnp.maximum(n, 1e-12)


def _sc_gather(table, idx_pos, idx_neg):
    NC, NS = 2, 16
    NW = NC * NS
    NPOS = idx_pos.shape[0]
    NNEG = idx_neg.shape[0]
    pos_w = NPOS // NW
    neg_w = NNEG // NW
    ch = 64
    mesh = plsc.VectorSubcoreMesh(core_axis_name="c", subcore_axis_name="s")

    @functools.partial(
        pl.kernel, mesh=mesh,
        out_type=[jax.ShapeDtypeStruct((NPOS, D), jnp.float32),
                  jax.ShapeDtypeStruct((NNEG, D), jnp.float32)],
        scratch_types=[
            pltpu.VMEM((pos_w,), jnp.int32),
            pltpu.VMEM((neg_w,), jnp.int32),
            pltpu.VMEM((pos_w, D), jnp.float32),
            pltpu.VMEM((ch, D), jnp.float32),
            pltpu.SemaphoreType.DMA,
            pltpu.SemaphoreType.DMA,
        ],
    )
    def gather_k(table_hbm, ip_hbm, in_hbm, op_hbm, on_hbm,
                 ipv, inv, pbuf, nbuf, psem, nsem):
        wid = lax.axis_index("s") * NC + lax.axis_index("c")
        pbase = wid * pos_w
        nbase = wid * neg_w
        pltpu.sync_copy(ip_hbm.at[pl.ds(pbase, pos_w)], ipv)
        pltpu.sync_copy(in_hbm.at[pl.ds(nbase, neg_w)], inv)
        pcp = pltpu.async_copy(table_hbm.at[ipv], pbuf, psem)

        def _neg(c, carry):
            off = pl.multiple_of(c * ch, ch)
            pltpu.async_copy(table_hbm.at[inv.at[pl.ds(off, ch)]],
                             nbuf, nsem).wait()
            pltpu.sync_copy(nbuf, on_hbm.at[pl.ds(nbase + off, ch)])
            return carry

        lax.fori_loop(0, neg_w // ch, _neg, 0)
        pcp.wait()
        pltpu.sync_copy(pbuf, op_hbm.at[pl.ds(pbase, pos_w)])

    return gather_k(table, idx_pos, idx_neg)


def kernel(text, gather_positive, gather_negative, label_repre,
           W1t, b1t, W2t, b2t, W1l, b1l, W2l, b2l):
    b1l2 = b1l.reshape(1, D)
    b2l2 = b2l.reshape(1, D)
    b1t2 = b1t.reshape(1, D)
    b2t2 = b2t.reshape(1, D)

    lab2d = pl.pallas_call(
        _label_mlp_body,
        out_shape=jax.ShapeDtypeStruct((L, D), jnp.float32),
    )(label_repre, W1l, b1l2, W2l, b2l2)

    rows_pos, rows_neg = _sc_gather(lab2d,
                                    gather_positive.reshape(-1),
                                    gather_negative.reshape(-1))
    label_positive = rows_pos.reshape(B, P, D)
    label_negative = rows_neg.reshape(B, NEG, D)

    DQ = D // 4
    wspecs = [pl.BlockSpec((KBLK, DQ), lambda k, j=j: (k, j))
              for j in range(4)]
    txt = pl.pallas_call(
        _text_mlp_body,
        grid=(NKB,),
        in_specs=[
            pl.BlockSpec((B, KBLK), lambda k: (0, k)),
            *wspecs,
            pl.BlockSpec((1, D), lambda k: (0, 0)),
            pl.BlockSpec((D, D), lambda k: (0, 0)),
            pl.BlockSpec((1, D), lambda k: (0, 0)),
        ],
        out_specs=pl.BlockSpec((B, D), lambda k: (0, 0)),
        out_shape=jax.ShapeDtypeStruct((B, D), jnp.float32),
        scratch_shapes=[pltpu.VMEM((B, D), jnp.float32)],
    )(text, W1t, W1t, W1t, W1t, b1t2, W2t, b2t2)

    return (txt, label_positive, label_negative)


# speedup vs baseline: 1.0354x; 1.0354x over previous
import functools

import jax
import jax.numpy as jnp
from jax import lax
from jax.experimental import pallas as pl
from jax.experimental.pallas import tpu as pltpu
from jax.experimental.pallas import tpu_sc as plsc

B = 128
L = 256
F_LAB = 512
D = 1024
P = 8
NEG = 32
TEXT_DIM = L * 128

NKB = 16
KBLK = TEXT_DIM // NKB


def _label_mlp_body(lab_ref, w1_ref, b1_ref, w2_ref, b2_ref, out_ref):
    h = jnp.dot(lab_ref[...].astype(jnp.bfloat16),
                w1_ref[...].astype(jnp.bfloat16),
                preferred_element_type=jnp.float32)
    h = jnp.maximum(h + b1_ref[...], 0.0)
    o = jnp.dot(h.astype(jnp.bfloat16), w2_ref[...].astype(jnp.bfloat16),
                preferred_element_type=jnp.float32)
    o = jnp.maximum(o + b2_ref[...], 0.0)
    n = jnp.sqrt(jnp.sum(o * o, axis=-1, keepdims=True))
    out_ref[...] = o / jnp.maximum(n, 1e-12)


def _text_mlp_body(x_ref, w1a_ref, w1b_ref, w1c_ref, w1d_ref,
                   b1_ref, w2_ref, b2_ref, out_ref, acc_ref):
    k = pl.program_id(0)

    @pl.when(k == 0)
    def _init():
        acc_ref[...] = jnp.zeros_like(acc_ref)

    xb = x_ref[...].astype(jnp.bfloat16)
    DQ = D // 4
    for j, wr in enumerate((w1a_ref, w1b_ref, w1c_ref, w1d_ref)):
        acc_ref[:, j * DQ:(j + 1) * DQ] += jnp.dot(
            xb, wr[...].astype(jnp.bfloat16),
            preferred_element_type=jnp.float32)

    @pl.when(k == NKB - 1)
    def _fin():
        h = jnp.maximum(acc_ref[...] + b1_ref[...], 0.0)
        o = jnp.dot(h, w2_ref[...], preferred_element_type=jnp.float32)
        o = jnp.maximum(o + b2_ref[...], 0.0)
        n = jnp.sqrt(jnp.sum(o * o, axis=-1, keepdims=True))
        out_ref[...] = o / jnp.maximum(n, 1e-12)


def _sc_gather(table, idx_pos, idx_neg):
    NC, NS = 2, 16
    NW = NC * NS
    rows_w = B // NW
    mesh = plsc.VectorSubcoreMesh(core_axis_name="c", subcore_axis_name="s")

    @functools.partial(
        pl.kernel, mesh=mesh,
        out_type=[jax.ShapeDtypeStruct((B * P, D), jnp.float32),
                  jax.ShapeDtypeStruct((B * NEG, D), jnp.float32)],
        scratch_types=[
            pltpu.VMEM((rows_w, P), jnp.int32),
            pltpu.VMEM((rows_w, NEG), jnp.int32),
            pltpu.VMEM((P, D), jnp.float32),
            pltpu.VMEM((NEG, D), jnp.float32),
            pltpu.SemaphoreType.DMA,
            pltpu.SemaphoreType.DMA,
        ],
    )
    def gather_k(table_hbm, ip_hbm, in_hbm, op_hbm, on_hbm,
                 ipv, inv, pbuf, nbuf, psem, nsem):
        wid = lax.axis_index("s") * NC + lax.axis_index("c")
        bbase = wid * rows_w
        pltpu.sync_copy(ip_hbm.at[pl.ds(bbase, rows_w)], ipv)
        pltpu.sync_copy(in_hbm.at[pl.ds(bbase, rows_w)], inv)
        for r in range(rows_w):
            g = bbase + r
            pcp = pltpu.async_copy(table_hbm.at[ipv.at[r]], pbuf, psem)
            ncp = pltpu.async_copy(table_hbm.at[inv.at[r]], nbuf, nsem)
            pcp.wait()
            pltpu.sync_copy(pbuf, op_hbm.at[pl.ds(g * P, P)])
            ncp.wait()
            pltpu.sync_copy(nbuf, on_hbm.at[pl.ds(g * NEG, NEG)])

    return gather_k(table, idx_pos, idx_neg)


def kernel(text, gather_positive, gather_negative, label_repre,
           W1t, b1t, W2t, b2t, W1l, b1l, W2l, b2l):
    b1l2 = b1l.reshape(1, D)
    b2l2 = b2l.reshape(1, D)
    b1t2 = b1t.reshape(1, D)
    b2t2 = b2t.reshape(1, D)

    lab2d = pl.pallas_call(
        _label_mlp_body,
        out_shape=jax.ShapeDtypeStruct((L, D), jnp.float32),
    )(label_repre, W1l, b1l2, W2l, b2l2)

    rows_pos, rows_neg = _sc_gather(lab2d, gather_positive, gather_negative)
    label_positive = rows_pos.reshape(B, P, D)
    label_negative = rows_neg.reshape(B, NEG, D)

    DQ = D // 4
    wspecs = [pl.BlockSpec((KBLK, DQ), lambda k, j=j: (k, j))
              for j in range(4)]
    txt = pl.pallas_call(
        _text_mlp_body,
        grid=(NKB,),
        in_specs=[
            pl.BlockSpec((B, KBLK), lambda k: (0, k)),
            *wspecs,
            pl.BlockSpec((1, D), lambda k: (0, 0)),
            pl.BlockSpec((D, D), lambda k: (0, 0)),
            pl.BlockSpec((1, D), lambda k: (0, 0)),
        ],
        out_specs=pl.BlockSpec((B, D), lambda k: (0, 0)),
        out_shape=jax.ShapeDtypeStruct((B, D), jnp.float32),
        scratch_shapes=[pltpu.VMEM((B, D), jnp.float32)],
    )(text, W1t, W1t, W1t, W1t, b1t2, W2t, b2t2)

    return (txt, label_positive, label_negative)
